# baseline (speedup 1.0000x reference)
import functools

import jax
import jax.numpy as jnp
from jax import lax
from jax.experimental import pallas as pl
from jax.experimental.pallas import tpu as pltpu
from jax.experimental.pallas import tpu_sc as plsc

NC = 2
NS = 16
L = 16
NW = NC * NS
N_TOK = 4 * 2048
D = 768
TOK_PER_W = N_TOK // NW
T = 16
NCHUNK = TOK_PER_W // T
DV = D // L

_mesh = plsc.VectorSubcoreMesh(core_axis_name="c", subcore_axis_name="s")


@functools.partial(
    pl.kernel,
    out_type=jax.ShapeDtypeStruct((N_TOK, D), jnp.float32),
    mesh=_mesh,
    scratch_types=[
        pltpu.VMEM((TOK_PER_W,), jnp.int32),
        pltpu.VMEM((TOK_PER_W,), jnp.int32),
        pltpu.VMEM((TOK_PER_W,), jnp.int32),
        pltpu.VMEM((TOK_PER_W,), jnp.int32),
        pltpu.VMEM((TOK_PER_W,), jnp.int32),
        pltpu.VMEM((T, D), jnp.float32),
        pltpu.VMEM((T, D), jnp.float32),
        pltpu.VMEM((T, D), jnp.float32),
        pltpu.VMEM((T, D), jnp.float32),
        pltpu.SemaphoreType.DMA,
        pltpu.SemaphoreType.DMA,
        pltpu.SemaphoreType.DMA,
        pltpu.SemaphoreType.DMA,
        pltpu.SemaphoreType.DMA,
    ],
)
def _embed4(ids_hbm, c0_hbm, c1_hbm, c2_hbm, c3_hbm,
            word_hbm, p0_hbm, p1_hbm, p2_hbm, p3_hbm,
            out_hbm, idsb, c0b, c1b, c2b, c3b,
            acc0, acc1, tmp0, tmp1,
            sem_w, sem_t0, sem_t1, sem_o0, sem_o1):
    wid = lax.axis_index("s") * NC + lax.axis_index("c")
    wbase = wid * TOK_PER_W

    pltpu.sync_copy(ids_hbm.at[pl.ds(wbase, TOK_PER_W)], idsb)
    w_first = pltpu.async_copy(
        word_hbm.at[idsb.at[pl.ds(0, T)]], acc0, sem_w)
    hs = [pltpu.async_copy(src.at[pl.ds(wbase, TOK_PER_W)], dst, sem_o0)
          for src, dst in ((c0_hbm, c0b), (c1_hbm, c1b),
                           (c2_hbm, c2b), (c3_hbm, c3b))]
    for h in hs:
        h.wait()

    accs = (acc0, acc1)
    sems_o = (sem_o0, sem_o1)

    def add_pass(accr, tmpr):
        def row(t, c):
            for j in range(DV):
                sl = pl.ds(j * L, L)
                plsc.addupdate(accr.at[t, sl], tmpr[t, sl])
            return c
        lax.fori_loop(0, T, row, 0)

    def chunk_body(i, a, first=False, w_pref=None):
        off = i * T
        gbase = wbase + off
        acc = accs[a]
        out_dst = out_hbm.at[pl.ds(gbase, T)]
        if not first:
            pltpu.make_async_copy(acc, out_dst, sems_o[a]).wait()
        if w_pref is None:
            w = pltpu.async_copy(word_hbm.at[idsb.at[pl.ds(off, T)]],
                                 acc, sem_w)
        else:
            w = w_pref
        g0 = pltpu.async_copy(p0_hbm.at[c0b.at[pl.ds(off, T)]], tmp0, sem_t0)
        g1 = pltpu.async_copy(p1_hbm.at[c1b.at[pl.ds(off, T)]], tmp1, sem_t1)
        w.wait()
        g0.wait()
        add_pass(acc, tmp0)
        g2 = pltpu.async_copy(p2_hbm.at[c2b.at[pl.ds(off, T)]], tmp0, sem_t0)
        g1.wait()
        add_pass(acc, tmp1)
        g3 = pltpu.async_copy(p3_hbm.at[c3b.at[pl.ds(off, T)]], tmp1, sem_t1)
        g2.wait()
        add_pass(acc, tmp0)
        g3.wait()
        add_pass(acc, tmp1)
        pltpu.async_copy(acc, out_dst, sems_o[a])

    chunk_body(0, 0, first=True, w_pref=w_first)
    chunk_body(1, 1, first=True)

    def loop_body(k, c):
        chunk_body(2 * k, 0)
        chunk_body(2 * k + 1, 1)
        return c

    lax.fori_loop(1, NCHUNK // 2, loop_body, 0)

    tail = wbase + (NCHUNK - 2) * T
    pltpu.make_async_copy(acc0, out_hbm.at[pl.ds(tail, T)], sem_o0).wait()
    pltpu.make_async_copy(acc1, out_hbm.at[pl.ds(tail + T, T)], sem_o1).wait()


def kernel(ids, coords, word, pos0, pos1, pos2, pos3):
    B, S = ids.shape
    ids_f = ids.reshape(N_TOK).astype(jnp.int32)
    c = coords.reshape(N_TOK, 4).astype(jnp.int32)
    out = _embed4(ids_f, c[:, 0], c[:, 1], c[:, 2], c[:, 3],
                  word, pos0, pos1, pos2, pos3)
    return out.reshape(B, S, D)


# speedup vs baseline: 1.1027x; 1.1027x over previous
import functools

import jax
import jax.numpy as jnp
from jax import lax
from jax.experimental import pallas as pl
from jax.experimental.pallas import tpu as pltpu
from jax.experimental.pallas import tpu_sc as plsc

NC = 2
NS = 16
L = 16
NW = NC * NS
N_TOK = 4 * 2048
D = 768
TOK_PER_W = N_TOK // NW
BUFT = 56
CHUNKS = (56, 56, 56, 56, 32)
DV = D // L

_mesh = plsc.VectorSubcoreMesh(core_axis_name="c", subcore_axis_name="s")


@functools.partial(
    pl.kernel,
    out_type=jax.ShapeDtypeStruct((N_TOK, D), jnp.float32),
    mesh=_mesh,
    scratch_types=[
        pltpu.VMEM((TOK_PER_W,), jnp.int32),
        pltpu.VMEM((TOK_PER_W,), jnp.int32),
        pltpu.VMEM((TOK_PER_W,), jnp.int32),
        pltpu.VMEM((TOK_PER_W,), jnp.int32),
        pltpu.VMEM((TOK_PER_W,), jnp.int32),
        pltpu.VMEM((BUFT, D), jnp.float32),
        pltpu.VMEM((BUFT, D), jnp.float32),
        pltpu.VMEM((BUFT, D), jnp.float32),
        pltpu.SemaphoreType.DMA,
        pltpu.SemaphoreType.DMA,
        pltpu.SemaphoreType.DMA,
        pltpu.SemaphoreType.DMA,
    ],
)
def _embed4(ids_hbm, c0_hbm, c1_hbm, c2_hbm, c3_hbm,
            word_hbm, p0_hbm, p1_hbm, p2_hbm, p3_hbm,
            out_hbm, idsb, c0b, c1b, c2b, c3b,
            acc, tmpa, tmpb,
            sem_w, sem_a, sem_b, sem_o):
    wid = lax.axis_index("s") * NC + lax.axis_index("c")
    wbase = wid * TOK_PER_W

    pltpu.sync_copy(ids_hbm.at[pl.ds(wbase, TOK_PER_W)], idsb)
    w_first = pltpu.async_copy(
        word_hbm.at[idsb.at[pl.ds(0, CHUNKS[0])]],
        acc.at[pl.ds(0, CHUNKS[0])], sem_w)
    hs = [pltpu.async_copy(src.at[pl.ds(wbase, TOK_PER_W)], dst, sem_o)
          for src, dst in ((c0_hbm, c0b), (c1_hbm, c1b),
                           (c2_hbm, c2b), (c3_hbm, c3b))]
    for h in hs:
        h.wait()

    def add_pass(accr, tmpr, rows):
        def row(t, c):
            for j in range(DV):
                sl = pl.ds(j * L, L)
                plsc.addupdate(accr.at[t, sl], tmpr[t, sl])
            return c
        lax.fori_loop(0, rows, row, 0)

    def final_pass(accr, tmpr, rows):
        def row(t, c):
            for j in range(DV):
                sl = pl.ds(j * L, L)
                tmpr[t, sl] = accr[t, sl] + tmpr[t, sl]
            return c
        lax.fori_loop(0, rows, row, 0)

    def chunk_body(off, n, prev=None, w_pref=None):
        acc_s = acc.at[pl.ds(0, n)]
        ta_s = tmpa.at[pl.ds(0, n)]
        tb_s = tmpb.at[pl.ds(0, n)]
        out_dst = out_hbm.at[pl.ds(wbase + off, n)]
        if w_pref is None:
            w = pltpu.async_copy(word_hbm.at[idsb.at[pl.ds(off, n)]],
                                 acc_s, sem_w)
        else:
            w = w_pref
        g0 = pltpu.async_copy(p0_hbm.at[c0b.at[pl.ds(off, n)]], ta_s, sem_a)
        if prev is not None:
            po, pn = prev
            pltpu.make_async_copy(tmpb.at[pl.ds(0, pn)],
                                  out_hbm.at[pl.ds(wbase + po, pn)],
                                  sem_o).wait()
        g1 = pltpu.async_copy(p1_hbm.at[c1b.at[pl.ds(off, n)]], tb_s, sem_b)
        w.wait()
        g0.wait()
        add_pass(acc, tmpa, n)
        g2 = pltpu.async_copy(p2_hbm.at[c2b.at[pl.ds(off, n)]], ta_s, sem_a)
        g1.wait()
        add_pass(acc, tmpb, n)
        g3 = pltpu.async_copy(p3_hbm.at[c3b.at[pl.ds(off, n)]], tb_s, sem_b)
        g2.wait()
        add_pass(acc, tmpa, n)
        g3.wait()
        final_pass(acc, tmpb, n)
        pltpu.async_copy(tb_s, out_dst, sem_o)

    off = 0
    prev = None
    for k, n in enumerate(CHUNKS):
        chunk_body(off, n, prev=prev, w_pref=w_first if k == 0 else None)
        prev = (off, n)
        off += n

    po, pn = prev
    pltpu.make_async_copy(tmpb.at[pl.ds(0, pn)],
                          out_hbm.at[pl.ds(wbase + po, pn)], sem_o).wait()


def kernel(ids, coords, word, pos0, pos1, pos2, pos3):
    B, S = ids.shape
    ids_f = ids.reshape(N_TOK).astype(jnp.int32)
    c = coords.reshape(N_TOK, 4).astype(jnp.int32)
    out = _embed4(ids_f, c[:, 0], c[:, 1], c[:, 2], c[:, 3],
                  word, pos0, pos1, pos2, pos3)
    return out.reshape(B, S, D)
